# baseline (speedup 1.0000x reference)
import jax
import jax.numpy as jnp
from jax import lax
from jax.experimental import pallas as pl
from jax.experimental.pallas import tpu as pltpu
from jax.experimental.pallas import tpu_sc as plsc

M = 400000
C = 128
R = 128
NUM_CHUNKS = M // R
NC = 2
NS = 16
NW = NC * NS
ITERS = -(-NUM_CHUNKS // NW)
NBUF = 4


def _sc_body(x_hbm, up_hbm, idxp_hbm, out_hbm, idx_v, up_v, lsem, gsem, ssem):
    wid = lax.axis_index("s") * NC + lax.axis_index("c")
    c0 = ITERS * wid
    ni = jnp.minimum(ITERS, NUM_CHUNKS - c0)

    pltpu.sync_copy(idxp_hbm.at[wid], idx_v)

    def base_of(k):
        return (c0 + k) * R

    def ldesc(k, b):
        return pltpu.make_async_copy(up_hbm.at[pl.ds(base_of(k), R)],
                                     up_v.at[b], lsem.at[b])

    def gdesc(k, b):
        return pltpu.make_async_copy(x_hbm.at[idx_v.at[k]], up_v.at[b],
                                     gsem.at[b])

    def sdesc(k, b):
        return pltpu.make_async_copy(up_v.at[b],
                                     out_hbm.at[pl.ds(base_of(k), R)],
                                     ssem.at[b])

    ldesc(0, 0).start()

    def step(j, carry):
        for b in range(NBUF):
            i = j * NBUF + b

            sb = (b - 1) % NBUF

            @pl.when((i - 1 >= 0) & (i - 1 < ni))
            def _():
                gdesc(i - 1, sb).wait()
                sdesc(i - 1, sb).start()

            lb = (b + 1) % NBUF

            @pl.when(i + 1 < ni)
            def _():
                @pl.when(i + 1 - NBUF >= 0)
                def _():
                    sdesc(i + 1 - NBUF, lb).wait()
                ldesc(i + 1, lb).start()

            @pl.when(i < ni)
            def _():
                ldesc(i, b).wait()
                gdesc(i, b).start(add=True)
        return carry

    lax.fori_loop(0, (ITERS + 1 + NBUF - 1) // NBUF, step, 0)

    for s in range(NBUF):
        sdesc(0, s).wait()


def kernel(x_features, up_x_features, unq_inv):
    idx = unq_inv.astype(jnp.int32)
    pad = (NW * ITERS * R) - M
    idxp = jnp.concatenate([idx, jnp.zeros((pad,), jnp.int32)])
    idxp = idxp.reshape(NW, ITERS, R)
    mesh = plsc.VectorSubcoreMesh(
        core_axis_name="c", subcore_axis_name="s",
        num_cores=NC, num_subcores=NS)
    f = pl.kernel(
        _sc_body,
        out_type=jax.ShapeDtypeStruct((M, C), jnp.float32),
        mesh=mesh,
        scratch_types=[
            pltpu.VMEM((ITERS, R), jnp.int32),
            pltpu.VMEM((NBUF, R, C), jnp.float32),
            pltpu.SemaphoreType.DMA((NBUF,)),
            pltpu.SemaphoreType.DMA((NBUF,)),
            pltpu.SemaphoreType.DMA((NBUF,)),
        ],
    )
    return f(x_features, up_x_features, idxp)


# speedup vs baseline: 1.0490x; 1.0490x over previous
import jax
import jax.numpy as jnp
from jax import lax
from jax.experimental import pallas as pl
from jax.experimental.pallas import tpu as pltpu
from jax.experimental.pallas import tpu_sc as plsc

M = 400000
C = 128
R = 128
NUM_CHUNKS = M // R
NC = 2
NS = 16
NW = NC * NS
ITERS = -(-NUM_CHUNKS // NW)
NBUF = 6
LEAD = 2


def _sc_body(x_hbm, up_hbm, idxp_hbm, out_hbm, idx_v, up_v, lsem, gsem, ssem):
    wid = lax.axis_index("s") * NC + lax.axis_index("c")
    c0 = ITERS * wid
    ni = jnp.minimum(ITERS, NUM_CHUNKS - c0)

    pltpu.sync_copy(idxp_hbm.at[wid], idx_v)

    def base_of(k):
        return (c0 + k) * R

    def ldesc(k, b):
        return pltpu.make_async_copy(up_hbm.at[pl.ds(base_of(k), R)],
                                     up_v.at[b], lsem.at[b])

    def gdesc(k, b):
        return pltpu.make_async_copy(x_hbm.at[idx_v.at[k]], up_v.at[b],
                                     gsem.at[b])

    def sdesc(k, b):
        return pltpu.make_async_copy(up_v.at[b],
                                     out_hbm.at[pl.ds(base_of(k), R)],
                                     ssem.at[b])

    for k in range(LEAD):
        @pl.when(k < ni)
        def _():
            ldesc(k, k).start()

    def step(j, carry):
        for b in range(NBUF):
            i = j * NBUF + b

            sb = (b - LEAD) % NBUF

            @pl.when((i - LEAD >= 0) & (i - LEAD < ni))
            def _():
                gdesc(i - LEAD, sb).wait()
                sdesc(i - LEAD, sb).start()

            lb = (b + LEAD) % NBUF

            @pl.when(i + LEAD < ni)
            def _():
                @pl.when(i + LEAD - NBUF >= 0)
                def _():
                    sdesc(i + LEAD - NBUF, lb).wait()
                ldesc(i + LEAD, lb).start()

            @pl.when(i < ni)
            def _():
                ldesc(i, b).wait()
                gdesc(i, b).start(add=True)
        return carry

    lax.fori_loop(0, (ITERS + LEAD + NBUF) // NBUF, step, 0)

    for s in range(NBUF):
        sdesc(0, s).wait()


def kernel(x_features, up_x_features, unq_inv):
    idx = unq_inv.astype(jnp.int32)
    pad = (NW * ITERS * R) - M
    idxp = jnp.concatenate([idx, jnp.zeros((pad,), jnp.int32)])
    idxp = idxp.reshape(NW, ITERS, R)
    mesh = plsc.VectorSubcoreMesh(
        core_axis_name="c", subcore_axis_name="s",
        num_cores=NC, num_subcores=NS)
    f = pl.kernel(
        _sc_body,
        out_type=jax.ShapeDtypeStruct((M, C), jnp.float32),
        mesh=mesh,
        scratch_types=[
            pltpu.VMEM((ITERS, R), jnp.int32),
            pltpu.VMEM((NBUF, R, C), jnp.float32),
            pltpu.SemaphoreType.DMA((NBUF,)),
            pltpu.SemaphoreType.DMA((NBUF,)),
            pltpu.SemaphoreType.DMA((NBUF,)),
        ],
    )
    return f(x_features, up_x_features, idxp)
